# baseline (speedup 1.0000x reference)
import jax
import jax.numpy as jnp
from jax import lax
from jax.experimental import pallas as pl
from jax.experimental.pallas import tpu as pltpu
from jax.experimental.pallas import tpu_sc as plsc

B = 16
MAX_LEN = 4096
TOTAL = 32768
D = 1024

NW = 32
ROWS_PER_W = (B * MAX_LEN) // NW
BLKS_PER_W = ROWS_PER_W // 8
HALVES = MAX_LEN // ROWS_PER_W
CB = 8
ZB = 32
CU_PAD = 32


def _issue_valid(flat5, out5, b, base_blk, a0, rv, q0, n, sem):
    if rv == 0:
        pltpu.async_copy(flat5.at[pl.ds(a0 + q0, n)],
                         out5.at[b, pl.ds(base_blk + q0, n)], sem)
    else:
        pltpu.async_copy(
            flat5.at[pl.ds(a0 + q0, n), :, pl.ds(rv, 8 - rv), :],
            out5.at[b, pl.ds(base_blk + q0, n), :, pl.ds(0, 8 - rv), :],
            sem)
        pltpu.async_copy(
            flat5.at[pl.ds(a0 + q0 + 1, n), :, pl.ds(0, rv), :],
            out5.at[b, pl.ds(base_blk + q0, n), :, pl.ds(8 - rv, rv), :],
            sem)


def _drain_valid(flat5, out5, b, rv, n, sem):
    if rv == 0:
        pltpu.make_async_copy(flat5.at[pl.ds(0, n)],
                              out5.at[b, pl.ds(0, n)], sem).wait()
    else:
        pltpu.make_async_copy(
            flat5.at[pl.ds(0, n), :, pl.ds(rv, 8 - rv), :],
            out5.at[b, pl.ds(0, n), :, pl.ds(0, 8 - rv), :], sem).wait()
        pltpu.make_async_copy(
            flat5.at[pl.ds(0, n), :, pl.ds(0, rv), :],
            out5.at[b, pl.ds(0, n), :, pl.ds(8 - rv, rv), :], sem).wait()


def _pad_stack_body(flat5, cu, zeros, out5, cu_v, zsh, csem, bit_sem,
                    zsem, zbit_sem, rowsem):
    c = lax.axis_index("c")
    s = lax.axis_index("s")
    wid = s * 2 + c
    b = wid // HALVES
    h = wid % HALVES
    p0 = h * ROWS_PER_W
    base_blk = h * BLKS_PER_W

    pltpu.sync_copy(cu, cu_v)
    cu_vec = cu_v[pl.ds(b, 16)]
    cu_b = cu_vec[0]
    cu_b1 = cu_vec[1]
    len_b = cu_b1 - cu_b
    v = jnp.clip(len_b - p0, 0, ROWS_PER_W)
    nbv = v // 8
    k = v - nbv * 8
    src = cu_b + p0
    r = lax.rem(src, 8)
    a0 = (src - r) // 8

    n_chunks = nbv // CB
    remb = nbv - n_chunks * CB

    for rv in range(8):
        @pl.when(r == rv)
        def _(rv=rv):
            def chunk(i, carry):
                _issue_valid(flat5, out5, b, base_blk, a0, rv, i * CB, CB,
                             csem)
                return carry

            lax.fori_loop(0, n_chunks, chunk, 0)
            rbase = n_chunks * CB
            for ki, n in enumerate((4, 2, 1)):
                shift = 2 - ki
                off = rbase + ((remb >> (shift + 1)) << (shift + 1))

                @pl.when((remb & n) != 0)
                def _(n=n, off=off):
                    _issue_valid(flat5, out5, b, base_blk, a0, rv, off, n,
                                 bit_sem.at[0])

    bv = base_blk + nbv

    def bnd_valid(i, carry):
        t = src + nbv * 8 + i
        tb = t // 8
        ti = lax.rem(t, 8)
        pltpu.async_copy(flat5.at[tb, :, pl.ds(ti, 1), :],
                         out5.at[b, bv, :, pl.ds(i, 1), :], rowsem)
        pltpu.make_async_copy(flat5.at[0, :, pl.ds(0, 1), :],
                              out5.at[b, 0, :, pl.ds(0, 1), :],
                              rowsem).wait()
        return carry

    def bnd_zero(i, carry):
        pltpu.async_copy(zsh.at[0, :, pl.ds(0, 1), :],
                         out5.at[b, bv, :, pl.ds(i, 1), :], rowsem)
        pltpu.make_async_copy(zsh.at[0, :, pl.ds(0, 1), :],
                              out5.at[b, 0, :, pl.ds(0, 1), :],
                              rowsem).wait()
        return carry

    zb0 = nbv + jnp.where(k > 0, 1, 0)
    nz = BLKS_PER_W - zb0
    nzc = nz // ZB
    zremb = nz - nzc * ZB

    pltpu.sync_copy(zeros, zsh.at[pl.ds(s * 2, 2)])
    plsc.subcore_barrier()

    lax.fori_loop(0, k, bnd_valid, 0)

    @pl.when(k > 0)
    def _():
        lax.fori_loop(k, 8, bnd_zero, 0)

    def zchunk(i, carry):
        pltpu.async_copy(zsh, out5.at[b, pl.ds(base_blk + zb0 + i * ZB, ZB)],
                         zsem)
        return carry

    lax.fori_loop(0, nzc, zchunk, 0)

    zrbase = zb0 + nzc * ZB
    for ki, n in enumerate((16, 8, 4, 2, 1)):
        shift = 4 - ki
        off = zrbase + ((zremb >> (shift + 1)) << (shift + 1))

        @pl.when((zremb & n) != 0)
        def _(n=n, off=off):
            pltpu.async_copy(zsh.at[pl.ds(0, n)],
                             out5.at[b, pl.ds(base_blk + off, n)],
                             zbit_sem.at[ki])

    for rv in range(8):
        @pl.when(r == rv)
        def _(rv=rv):
            def dchunk(i, carry):
                _drain_valid(flat5, out5, b, rv, CB, csem)
                return carry

            lax.fori_loop(0, n_chunks, dchunk, 0)
            for n in (4, 2, 1):
                @pl.when((remb & n) != 0)
                def _(n=n):
                    _drain_valid(flat5, out5, b, rv, n, bit_sem.at[0])

    def dzchunk(i, carry):
        pltpu.make_async_copy(zsh, out5.at[b, pl.ds(0, ZB)], zsem).wait()
        return carry

    lax.fori_loop(0, nzc, dzchunk, 0)

    for ki, n in enumerate((16, 8, 4, 2, 1)):
        @pl.when((zremb & n) != 0)
        def _(n=n, ki=ki):
            pltpu.make_async_copy(zsh.at[pl.ds(0, n)],
                                  out5.at[b, pl.ds(0, n)],
                                  zbit_sem.at[ki]).wait()


_mesh = plsc.VectorSubcoreMesh(core_axis_name="c", subcore_axis_name="s",
                               num_cores=2, num_subcores=16)

_pad_stack = pl.kernel(
    _pad_stack_body,
    out_type=jax.ShapeDtypeStruct((B, MAX_LEN // 8, 8, 8, 128), jnp.float32),
    mesh=_mesh,
    scratch_types=[
        pltpu.VMEM((CU_PAD,), jnp.int32),
        pltpu.VMEM_SHARED((ZB, 8, 8, 128), jnp.float32),
        pltpu.SemaphoreType.DMA,
        pltpu.SemaphoreType.DMA((1,)),
        pltpu.SemaphoreType.DMA,
        pltpu.SemaphoreType.DMA((5,)),
        pltpu.SemaphoreType.DMA,
    ],
    compiler_params=pltpu.CompilerParams(use_tc_tiling_on_sc=False),
)


def kernel(flat, cu_seqlens):
    cu_pad = jnp.zeros((CU_PAD,), jnp.int32).at[: B + 1].set(
        cu_seqlens.astype(jnp.int32))
    zeros = jnp.zeros((2, 8, 8, 128), jnp.float32)
    flat5 = flat.reshape(TOTAL // 8, 8, 8, 128).transpose(0, 2, 1, 3)
    out5 = _pad_stack(flat5, cu_pad, zeros)
    return out5.transpose(0, 1, 3, 2, 4).reshape(B, MAX_LEN, D)


# speedup vs baseline: 22.1471x; 22.1471x over previous
import jax
import jax.numpy as jnp
from jax import lax
from jax.experimental import pallas as pl
from jax.experimental.pallas import tpu as pltpu
from jax.experimental.pallas import tpu_sc as plsc

B = 16
MAX_LEN = 4096
TOTAL = 32768
D = 1024

NW = 32
ROWS_PER_W = (B * MAX_LEN) // NW
BLKS_PER_W = ROWS_PER_W // 8
NBLK = TOTAL // 8
HALVES = MAX_LEN // ROWS_PER_W
CB = 4
NBUF = 3
ZB = 8
CU_PAD = 32
CBITS = (2, 1)
ZBITS = (4, 2, 1)


def _issue_writes(bufslot, out5, b, dst_blk, dbl, rv, n, sem):
    if rv == 0:
        pltpu.async_copy(bufslot.at[pl.ds(dbl, n)],
                         out5.at[b, pl.ds(dst_blk, n)], sem)
    else:
        pltpu.async_copy(
            bufslot.at[pl.ds(dbl, n), :, pl.ds(rv, 8 - rv), :],
            out5.at[b, pl.ds(dst_blk, n), :, pl.ds(0, 8 - rv), :], sem)
        pltpu.async_copy(
            bufslot.at[pl.ds(dbl + 1, n), :, pl.ds(0, rv), :],
            out5.at[b, pl.ds(dst_blk, n), :, pl.ds(8 - rv, rv), :], sem)


def _wait_writes(bufslot, out5, b, rv, n, sem):
    if rv == 0:
        pltpu.make_async_copy(bufslot.at[pl.ds(0, n)],
                              out5.at[b, pl.ds(0, n)], sem).wait()
    else:
        pltpu.make_async_copy(
            bufslot.at[pl.ds(0, n), :, pl.ds(rv, 8 - rv), :],
            out5.at[b, pl.ds(0, n), :, pl.ds(0, 8 - rv), :], sem).wait()
        pltpu.make_async_copy(
            bufslot.at[pl.ds(1, n), :, pl.ds(0, rv), :],
            out5.at[b, pl.ds(0, n), :, pl.ds(8 - rv, rv), :], sem).wait()


def _pad_stack_body(flat5, cu, zeros, out5, cu_v, buf, zsh,
                    rsem, wsem, bitw_sem, rowsem, zsem, zbit_sem):
    c = lax.axis_index("c")
    s = lax.axis_index("s")
    wid = s * 2 + c
    b = wid // HALVES
    h = wid % HALVES
    p0 = h * ROWS_PER_W
    base_blk = h * BLKS_PER_W

    pltpu.sync_copy(cu, cu_v)

    @pl.when(s < ZB)
    def _():
        pltpu.sync_copy(zeros, zsh.at[pl.ds(s, 1)])

    plsc.subcore_barrier()

    cu_vec = cu_v[pl.ds(b, 16)]
    cu_b = cu_vec[0]
    cu_b1 = cu_vec[1]
    len_b = cu_b1 - cu_b
    v = jnp.clip(len_b - p0, 0, ROWS_PER_W)
    nbv = v // 8
    k = v - nbv * 8
    src = cu_b + p0
    r = lax.rem(src, 8)
    a0 = (src - r) // 8
    n_chunks = nbv // CB
    remb = nbv - n_chunks * CB

    zb0 = nbv + jnp.where(k > 0, 1, 0)
    nz = BLKS_PER_W - zb0
    nzc = nz // ZB
    zremb = nz - nzc * ZB

    def zchunk(i, carry):
        pltpu.async_copy(zsh, out5.at[b, pl.ds(base_blk + zb0 + i * ZB, ZB)],
                         zsem)
        return carry

    lax.fori_loop(0, nzc, zchunk, 0)

    zrbase = zb0 + nzc * ZB
    for ki, n in enumerate(ZBITS):
        shift = len(ZBITS) - 1 - ki
        off = zrbase + ((zremb >> (shift + 1)) << (shift + 1))

        @pl.when((zremb & n) != 0)
        def _(n=n, off=off, ki=ki):
            pltpu.async_copy(zsh.at[pl.ds(0, n)],
                             out5.at[b, pl.ds(base_blk + off, n)],
                             zbit_sem.at[ki])

    def read_chunk(q0, nread, dst, sem):
        st = jnp.minimum(a0 + q0, NBLK - nread)
        pltpu.async_copy(flat5.at[pl.ds(st, nread)], dst, sem)

    for pb in range(2):
        @pl.when(pb < n_chunks)
        def _(pb=pb):
            read_chunk(pb * CB, CB + 1, buf.at[pb], rsem.at[pb])

    def copy_step(i, carry):
        slot = lax.rem(i, NBUF)
        pltpu.make_async_copy(flat5.at[pl.ds(0, CB + 1)], buf.at[slot],
                              rsem.at[slot]).wait()
        st = jnp.minimum(a0 + i * CB, NBLK - (CB + 1))
        dbl = (a0 + i * CB) - st
        for rv in range(8):
            @pl.when(r == rv)
            def _(rv=rv):
                _issue_writes(buf.at[slot], out5, b, base_blk + i * CB,
                              dbl, rv, CB, wsem.at[slot])
        nxt = i + 2
        ws = lax.rem(nxt, NBUF)

        @pl.when(nxt < n_chunks)
        def _():
            @pl.when(i >= 1)
            def _():
                for rv in range(8):
                    @pl.when(r == rv)
                    def _(rv=rv):
                        _wait_writes(buf.at[ws], out5, b, rv, CB,
                                     wsem.at[ws])

            read_chunk(nxt * CB, CB + 1, buf.at[ws], rsem.at[ws])

        return carry

    lax.fori_loop(0, n_chunks, copy_step, 0)

    for sl in range(NBUF):
        @pl.when(sl < n_chunks)
        def _(sl=sl):
            for rv in range(8):
                @pl.when(r == rv)
                def _(rv=rv, sl=sl):
                    _wait_writes(buf.at[sl], out5, b, rv, CB, wsem.at[sl])

    rbase = n_chunks * CB
    for ki, n in enumerate(CBITS):
        shift = len(CBITS) - 1 - ki
        off = rbase + ((remb >> (shift + 1)) << (shift + 1))

        @pl.when((remb & n) != 0)
        def _(n=n, off=off, ki=ki):
            sl = ki % NBUF
            read_chunk(off, n + 1, buf.at[sl, pl.ds(0, n + 1)], rsem.at[sl])
            pltpu.make_async_copy(flat5.at[pl.ds(0, n + 1)],
                                  buf.at[sl, pl.ds(0, n + 1)],
                                  rsem.at[sl]).wait()
            st = jnp.minimum(a0 + off, NBLK - (n + 1))
            dbl = (a0 + off) - st
            for rv in range(8):
                @pl.when(r == rv)
                def _(rv=rv):
                    _issue_writes(buf.at[sl], out5, b, base_blk + off,
                                  dbl, rv, n, bitw_sem.at[ki])

    for ki, n in enumerate(CBITS):
        @pl.when((remb & n) != 0)
        def _(n=n, ki=ki):
            for rv in range(8):
                @pl.when(r == rv)
                def _(rv=rv, n=n, ki=ki):
                    _wait_writes(buf.at[ki % NBUF], out5, b, rv, n,
                                 bitw_sem.at[ki])

    bv = base_blk + nbv

    @pl.when(k > 0)
    def _():
        def bnd_row(i, carry):
            t = src + nbv * 8 + i
            tb = t // 8
            ti = lax.rem(t, 8)
            pltpu.async_copy(flat5.at[tb, :, pl.ds(ti, 1), :],
                             out5.at[b, bv, :, pl.ds(i, 1), :], rowsem)
            return carry

        lax.fori_loop(0, k, bnd_row, 0)

        def bnd_zrow(i, carry):
            pltpu.async_copy(zeros.at[0, :, pl.ds(0, 1), :],
                             out5.at[b, bv, :, pl.ds(i, 1), :], rowsem)
            return carry

        lax.fori_loop(k, 8, bnd_zrow, 0)

        def bnd_drain(i, carry):
            pltpu.make_async_copy(zeros.at[0, :, pl.ds(0, 1), :],
                                  out5.at[b, 0, :, pl.ds(0, 1), :],
                                  rowsem).wait()
            return carry

        lax.fori_loop(0, 8, bnd_drain, 0)

    def dzchunk(i, carry):
        pltpu.make_async_copy(zsh, out5.at[b, pl.ds(0, ZB)], zsem).wait()
        return carry

    lax.fori_loop(0, nzc, dzchunk, 0)

    for ki, n in enumerate(ZBITS):
        @pl.when((zremb & n) != 0)
        def _(n=n, ki=ki):
            pltpu.make_async_copy(zsh.at[pl.ds(0, n)],
                                  out5.at[b, pl.ds(0, n)],
                                  zbit_sem.at[ki]).wait()


_mesh = plsc.VectorSubcoreMesh(core_axis_name="c", subcore_axis_name="s",
                               num_cores=2, num_subcores=16)

_pad_stack = pl.kernel(
    _pad_stack_body,
    out_type=jax.ShapeDtypeStruct((B, MAX_LEN // 8, 8, 8, 128), jnp.float32),
    mesh=_mesh,
    scratch_types=[
        pltpu.VMEM((CU_PAD,), jnp.int32),
        pltpu.VMEM((NBUF, CB + 1, 8, 8, 128), jnp.float32),
        pltpu.VMEM_SHARED((ZB, 8, 8, 128), jnp.float32),
        pltpu.SemaphoreType.DMA((NBUF,)),
        pltpu.SemaphoreType.DMA((NBUF,)),
        pltpu.SemaphoreType.DMA((len(CBITS),)),
        pltpu.SemaphoreType.DMA,
        pltpu.SemaphoreType.DMA,
        pltpu.SemaphoreType.DMA((len(ZBITS),)),
    ],
    compiler_params=pltpu.CompilerParams(use_tc_tiling_on_sc=False),
)


def kernel(flat, cu_seqlens):
    cu_pad = jnp.zeros((CU_PAD,), jnp.int32).at[: B + 1].set(
        cu_seqlens.astype(jnp.int32))
    zeros = jnp.zeros((1, 8, 8, 128), jnp.float32)
    flat5 = flat.reshape(TOTAL // 8, 8, 8, 128).transpose(0, 2, 1, 3)
    out5 = _pad_stack(flat5, cu_pad, zeros)
    return out5.transpose(0, 1, 3, 2, 4).reshape(B, MAX_LEN, D)
